# baseline (speedup 1.0000x reference)
import jax
import jax.numpy as jnp
from jax.experimental import pallas as pl

_ROWS = 512


def _frag_encode_body(a_ref, w_ref, o_ref):
    a = a_ref[...]
    rows, cols = a.shape
    idx = jnp.argmax(a, axis=1).astype(jnp.int32)
    col = jax.lax.broadcasted_iota(jnp.int32, (rows, cols), 1)
    onehot = (col == idx[:, None]).astype(jnp.float32)
    o_ref[...] = jnp.dot(onehot, w_ref[...], preferred_element_type=jnp.float32)


def kernel(frag_attr, embedding_weight):
    n, c = frag_attr.shape
    _, d = embedding_weight.shape
    return pl.pallas_call(
        _frag_encode_body,
        grid=(n // _ROWS,),
        in_specs=[
            pl.BlockSpec((_ROWS, c), lambda i: (i, 0)),
            pl.BlockSpec((c, d), lambda i: (0, 0)),
        ],
        out_specs=pl.BlockSpec((_ROWS, d), lambda i: (i, 0)),
        out_shape=jax.ShapeDtypeStruct((n, d), jnp.float32),
    )(frag_attr, embedding_weight)
